# baseline (speedup 1.0000x reference)
import functools

import jax
import jax.numpy as jnp
from jax import lax
from jax.experimental import pallas as pl
from jax.experimental.pallas import tpu as pltpu
from jax.experimental.pallas import tpu_sc as plsc

N = 10000
E = 320000
DF = 128
DE = 16
H = 64
L = 32

H2 = 128
HS = 80
NC = 2
NS = 16
NW = NC * NS
EPW = E // NW
CB = 400
NCH = EPW // CB
NPAIR = (NCH - 1) // 2
ZR = 624
ZREM = N - ZR * NS

_SC_MESH = plsc.VectorSubcoreMesh(core_axis_name="c", subcore_axis_name="s")
_SC_PARAMS = pltpu.CompilerParams(use_tc_tiling_on_sc=False)


def _sliced_copy(src, dst, sid):
    pltpu.sync_copy(src.at[pl.ds(sid * ZR, ZR)], dst.at[pl.ds(sid * ZR, ZR)])

    @pl.when(sid == NS - 1)
    def _():
        pltpu.sync_copy(src.at[pl.ds(ZR * NS, ZREM)], dst.at[pl.ds(ZR * NS, ZREM)])


def _sc_edge_static(eh_half, dst1, zeros_hs, half):
    EPW2 = E // 2 // NW
    CB2 = 200
    NCH2 = EPW2 // CB2
    NPAIR2 = (NCH2 - 1) // 2

    @functools.partial(
        pl.kernel,
        out_type=jax.ShapeDtypeStruct((NC, N, HS), jnp.float32),
        mesh=_SC_MESH,
        compiler_params=_SC_PARAMS,
        scratch_types=[
            pltpu.VMEM((EPW2,), jnp.int32),
            pltpu.VMEM((CB2, HS), jnp.float32),
            pltpu.VMEM((CB2, HS), jnp.float32),
            pltpu.VMEM_SHARED((N, HS), jnp.float32),
            pltpu.SemaphoreType.DMA,
            pltpu.SemaphoreType.DMA,
        ],
    )
    def k(eh_h, dst_h, z_h, seh_o, idx_v, rows_a, rows_b, seh_sh,
          sem_a, sem_b):
        cid = lax.axis_index("c")
        sid = lax.axis_index("s")
        wid = cid * NS + sid
        _sliced_copy(z_h, seh_sh, sid)
        base_w = wid * EPW2
        pltpu.sync_copy(dst_h.at[pl.ds(half * (E // 2) + base_w, EPW2)],
                        idx_v)
        plsc.subcore_barrier()

        def load(j, buf, sem):
            return pltpu.async_copy(
                eh_h.at[pl.ds(base_w + j * CB2, CB2), pl.ds(0, HS)], buf, sem)

        def flush(j, buf):
            pltpu.sync_copy(buf, seh_sh.at[idx_v.at[pl.ds(j * CB2, CB2)]],
                            add=True)

        load(0, rows_a, sem_a).wait()

        def pair(i, carry):
            ja = 2 * i
            db = load(ja + 1, rows_b, sem_b)
            flush(ja, rows_a)
            da = load(ja + 2, rows_a, sem_a)
            db.wait()
            flush(ja + 1, rows_b)
            da.wait()
            return carry

        lax.fori_loop(0, NPAIR2, pair, 0)
        flush(NCH2 - 1, rows_a)
        plsc.subcore_barrier()
        _sliced_copy(seh_sh, seh_o.at[cid], sid)

    return k(eh_half, dst1, zeros_hs)


def _sc_gather_segsum(h, src3, dst3, zeros_h):
    @functools.partial(
        pl.kernel,
        out_type=jax.ShapeDtypeStruct((NC, N, H), jnp.float32),
        mesh=_SC_MESH,
        compiler_params=_SC_PARAMS,
        scratch_types=[
            pltpu.VMEM((EPW,), jnp.int32),
            pltpu.VMEM((EPW,), jnp.int32),
            pltpu.VMEM((CB, H), jnp.float32),
            pltpu.VMEM((CB, H), jnp.float32),
            pltpu.VMEM_SHARED((N, H), jnp.float32),
            pltpu.SemaphoreType.DMA,
            pltpu.SemaphoreType.DMA,
        ],
    )
    def k(h_h, src_h, dst_h, z_h, s_o, idxs_v, idxd_v, rows_a, rows_b,
          acc_sh, sem_a, sem_b):
        cid = lax.axis_index("c")
        sid = lax.axis_index("s")
        wid = cid * NS + sid
        _sliced_copy(z_h, acc_sh, sid)
        base_w = wid * EPW
        pltpu.sync_copy(src_h.at[pl.ds(base_w, EPW)], idxs_v)
        pltpu.sync_copy(dst_h.at[pl.ds(base_w, EPW)], idxd_v)
        plsc.subcore_barrier()

        def gather(j, buf, sem):
            return pltpu.async_copy(h_h.at[idxs_v.at[pl.ds(j * CB, CB)]],
                                    buf, sem)

        def flush(j, buf):
            pltpu.sync_copy(buf, acc_sh.at[idxd_v.at[pl.ds(j * CB, CB)]],
                            add=True)

        gather(0, rows_a, sem_a).wait()

        def pair(i, carry):
            ja = 2 * i
            db = gather(ja + 1, rows_b, sem_b)
            flush(ja, rows_a)
            da = gather(ja + 2, rows_a, sem_a)
            db.wait()
            flush(ja + 1, rows_b)
            da.wait()
            return carry

        lax.fori_loop(0, NPAIR, pair, 0)
        flush(NCH - 1, rows_a)
        plsc.subcore_barrier()
        _sliced_copy(acc_sh, s_o.at[cid], sid)

    return k(h, src3, dst3, zeros_h)


def _sc_gather_z(z, src3, dst3):
    @functools.partial(
        pl.kernel,
        out_type=(
            jax.ShapeDtypeStruct((NW * NCH, CB, L), jnp.float32),
            jax.ShapeDtypeStruct((NW * NCH, CB, L), jnp.float32),
        ),
        mesh=_SC_MESH,
        compiler_params=_SC_PARAMS,
        scratch_types=[
            pltpu.VMEM((EPW,), jnp.int32),
            pltpu.VMEM((EPW,), jnp.int32),
            pltpu.VMEM((CB, L), jnp.float32),
            pltpu.VMEM((CB, L), jnp.float32),
            pltpu.VMEM((CB, L), jnp.float32),
            pltpu.VMEM((CB, L), jnp.float32),
            pltpu.SemaphoreType.DMA,
            pltpu.SemaphoreType.DMA,
        ],
    )
    def k(z_h, src_h, dst_h, gzs_o, gzd_o, idxs_v, idxd_v,
          rs_a, rd_a, rs_b, rd_b, sem_a, sem_b):
        cid = lax.axis_index("c")
        sid = lax.axis_index("s")
        wid = cid * NS + sid
        base_w = wid * EPW
        pltpu.sync_copy(src_h.at[pl.ds(base_w, EPW)], idxs_v)
        pltpu.sync_copy(dst_h.at[pl.ds(base_w, EPW)], idxd_v)

        def gathers(j, rs, rd, sem):
            sl = pl.ds(j * CB, CB)
            cs = pltpu.async_copy(z_h.at[idxs_v.at[sl]], rs, sem)
            cd = pltpu.async_copy(z_h.at[idxd_v.at[sl]], rd, sem)
            return cs, cd

        def flush(j, rs, rd):
            pltpu.sync_copy(rs, gzs_o.at[wid * NCH + j])
            pltpu.sync_copy(rd, gzd_o.at[wid * NCH + j])

        ca, cb2 = gathers(0, rs_a, rd_a, sem_a)
        ca.wait()
        cb2.wait()

        def pair(i, carry):
            ja = 2 * i
            b1, b2 = gathers(ja + 1, rs_b, rd_b, sem_b)
            flush(ja, rs_a, rd_a)
            a1, a2 = gathers(ja + 2, rs_a, rd_a, sem_a)
            b1.wait()
            b2.wait()
            flush(ja + 1, rs_b, rd_b)
            a1.wait()
            a2.wait()
            return carry

        lax.fori_loop(0, NPAIR, pair, 0)
        flush(NCH - 1, rs_a, rd_a)

    return k(z, src3, dst3)


def _tc_in_proj(x, w, b):
    BM = 2000

    def body(x_r, w_r, b_r, o_r):
        o_r[...] = jax.nn.relu(
            jnp.dot(x_r[...], w_r[...], preferred_element_type=jnp.float32)
            + b_r[...]
        )

    return pl.pallas_call(
        body,
        grid=(N // BM,),
        in_specs=[
            pl.BlockSpec((BM, DF), lambda i: (i, 0)),
            pl.BlockSpec((DF, H), lambda i: (0, 0)),
            pl.BlockSpec((1, H), lambda i: (0, 0)),
        ],
        out_specs=pl.BlockSpec((BM, H), lambda i: (i, 0)),
        out_shape=jax.ShapeDtypeStruct((N, H), jnp.float32),
    )(x, w, b)


def _tc_edge_proj(ea_t, w_pad, b_pad):
    BM = 6400
    nblk = E // 2 // BM

    def body(a_r, w_r, b_r, o_r):
        o_r[...] = jax.nn.relu(
            lax.dot_general(a_r[...], w_r[...], (((0,), (0,)), ((), ())),
                            preferred_element_type=jnp.float32)
            + b_r[...]
        )

    def one_half(half):
        return pl.pallas_call(
            body,
            grid=(nblk,),
            in_specs=[
                pl.BlockSpec((DE, BM), lambda i: (0, i + half * nblk)),
                pl.BlockSpec((DE, H2), lambda i: (0, 0)),
                pl.BlockSpec((1, H2), lambda i: (0, 0)),
            ],
            out_specs=pl.BlockSpec((BM, H2), lambda i: (i, 0)),
            out_shape=jax.ShapeDtypeStruct((E // 2, H2), jnp.float32),
        )(ea_t, w_pad, b_pad)

    return one_half(0), one_half(1)


def _tc_layer_combine(h, s, seh1, seh2, ws, wn, b):
    def body(h_r, s_r, e1_r, e2_r, ws_r, wn_r, b_r, o_r):
        deg = (e1_r[0, :, H:H + 1] + e1_r[1, :, H:H + 1]
               + e2_r[0, :, H:H + 1] + e2_r[1, :, H:H + 1])
        deg = jnp.maximum(deg, 1.0)
        agg = (s_r[0] + s_r[1] + e1_r[0, :, :H] + e1_r[1, :, :H]
               + e2_r[0, :, :H] + e2_r[1, :, :H]) / deg
        o_r[...] = jax.nn.relu(
            jnp.dot(h_r[...], ws_r[...], preferred_element_type=jnp.float32)
            + jnp.dot(agg, wn_r[...], preferred_element_type=jnp.float32)
            + b_r[...]
        )

    return pl.pallas_call(
        body,
        out_shape=jax.ShapeDtypeStruct((N, H), jnp.float32),
    )(h, s, seh1, seh2, ws, wn, b)


def _tc_latent_node(h, w_lat, b_lat, nd_w1, nd_b1, nd_w2, nd_b2):
    def body(h_r, wl_r, bl_r, w1_r, b1_r, w2_r, b2_r, z_o, g_o, rn_o):
        z = jnp.dot(h_r[...], wl_r[...], preferred_element_type=jnp.float32) + bl_r[...]
        z_o[...] = z
        g_o[...] = jnp.mean(z, axis=0, keepdims=True)
        hid = jax.nn.relu(
            jnp.dot(z, w1_r[...], preferred_element_type=jnp.float32) + b1_r[...]
        )
        rn_o[...] = (
            jnp.dot(hid, w2_r[...], preferred_element_type=jnp.float32) + b2_r[...]
        )

    return pl.pallas_call(
        body,
        out_shape=(
            jax.ShapeDtypeStruct((N, L), jnp.float32),
            jax.ShapeDtypeStruct((1, L), jnp.float32),
            jax.ShapeDtypeStruct((N, DF), jnp.float32),
        ),
    )(h, w_lat, b_lat, nd_w1, nd_b1, nd_w2, nd_b2)


def _tc_edge_decode(gzs4, gzd4, w1a4, w1b4, b1_4, w2_4, b2_4):
    BM4 = 4000

    def body(s_r, d_r, w1a_r, w1b_r, b1_r, w2_r, b2_r, o_r):
        hid = jax.nn.relu(
            jnp.dot(s_r[...], w1a_r[...], preferred_element_type=jnp.float32)
            + jnp.dot(d_r[...], w1b_r[...], preferred_element_type=jnp.float32)
            + b1_r[...]
        )
        o_r[...] = (
            jnp.dot(hid, w2_r[...], preferred_element_type=jnp.float32) + b2_r[...]
        )

    return pl.pallas_call(
        body,
        grid=(E // 4 // BM4,),
        in_specs=[
            pl.BlockSpec((BM4, 4 * L), lambda i: (i, 0)),
            pl.BlockSpec((BM4, 4 * L), lambda i: (i, 0)),
            pl.BlockSpec((4 * L, 4 * H), lambda i: (0, 0)),
            pl.BlockSpec((4 * L, 4 * H), lambda i: (0, 0)),
            pl.BlockSpec((1, 4 * H), lambda i: (0, 0)),
            pl.BlockSpec((4 * H, 4 * DE), lambda i: (0, 0)),
            pl.BlockSpec((1, 4 * DE), lambda i: (0, 0)),
        ],
        out_specs=pl.BlockSpec((BM4, 4 * DE), lambda i: (i, 0)),
        out_shape=jax.ShapeDtypeStruct((E // 4, 4 * DE), jnp.float32),
    )(gzs4, gzd4, w1a4, w1b4, b1_4, w2_4, b2_4)


def kernel(x, edge_index, edge_attr, W_in, b_in, W_edge, b_edge, W_self0,
           W_nbr0, b0, W_self1, W_nbr1, b1, W_lat, b_lat, nd_W1, nd_b1,
           nd_W2, nd_b2, ed_W1, ed_b1, ed_W2, ed_b2):
    src = edge_index[0].astype(jnp.int32)
    dst = edge_index[1].astype(jnp.int32)

    zeros_h = jnp.zeros((N, H), jnp.float32)
    zeros_hs = jnp.zeros((N, HS), jnp.float32)

    w_edge_pad = jnp.zeros((DE, H2), jnp.float32).at[:, :H].set(W_edge)
    b_edge_pad = jnp.zeros((1, H2), jnp.float32).at[:, :H].set(b_edge)
    b_edge_pad = b_edge_pad.at[:, H].set(1.0)
    bd = jax.scipy.linalg.block_diag
    w1a4 = bd(*([ed_W1[:L]] * 4))
    w1b4 = bd(*([ed_W1[L:]] * 4))
    w2_4 = bd(*([ed_W2] * 4))
    b1_4 = jnp.tile(ed_b1, 4).reshape(1, 4 * H)
    b2_4 = jnp.tile(ed_b2, 4).reshape(1, 4 * DE)

    h = _tc_in_proj(x, W_in, b_in.reshape(1, H))
    eh1, eh2 = _tc_edge_proj(edge_attr.T, w_edge_pad, b_edge_pad)

    seh1 = _sc_edge_static(eh1, dst, zeros_hs, 0)
    seh2 = _sc_edge_static(eh2, dst, zeros_hs, 1)

    for ws, wn, b in ((W_self0, W_nbr0, b0), (W_self1, W_nbr1, b1)):
        s = _sc_gather_segsum(h, src, dst, zeros_h)
        h = _tc_layer_combine(h, s, seh1, seh2, ws, wn, b.reshape(1, H))

    z, g, recon_node = _tc_latent_node(
        h, W_lat, b_lat.reshape(1, L), nd_W1, nd_b1.reshape(1, H),
        nd_W2, nd_b2.reshape(1, DF))

    gzs, gzd = _sc_gather_z(z, src, dst)
    recon_edge4 = _tc_edge_decode(
        gzs.reshape(E // 4, 4 * L), gzd.reshape(E // 4, 4 * L),
        w1a4, w1b4, b1_4, w2_4, b2_4)

    recon_edge = recon_edge4.reshape(E // 4, 4, DE).transpose(2, 0, 1).reshape(DE, E).T

    return (z, g, recon_node, recon_edge)


# speedup vs baseline: 1.0305x; 1.0305x over previous
import functools

import jax
import jax.numpy as jnp
from jax import lax
from jax.experimental import pallas as pl
from jax.experimental.pallas import tpu as pltpu
from jax.experimental.pallas import tpu_sc as plsc

N = 10000
E = 320000
DF = 128
DE = 16
H = 64
L = 32

H2 = 128
HS = 80
NC = 2
NS = 16
NW = NC * NS
EPW = E // NW
CB = 400
NCH = EPW // CB
NPAIR = (NCH - 1) // 2
ZR = 624
ZREM = N - ZR * NS

_SC_MESH = plsc.VectorSubcoreMesh(core_axis_name="c", subcore_axis_name="s")
_SC_PARAMS = pltpu.CompilerParams(use_tc_tiling_on_sc=False)


def _sliced_copy(src, dst, sid):
    pltpu.sync_copy(src.at[pl.ds(sid * ZR, ZR)], dst.at[pl.ds(sid * ZR, ZR)])

    @pl.when(sid == NS - 1)
    def _():
        pltpu.sync_copy(src.at[pl.ds(ZR * NS, ZREM)], dst.at[pl.ds(ZR * NS, ZREM)])


def _sc_edge_static(eh, dst3, zeros_hs):
    @functools.partial(
        pl.kernel,
        out_type=jax.ShapeDtypeStruct((NC, N, HS), jnp.float32),
        mesh=_SC_MESH,
        compiler_params=_SC_PARAMS,
        scratch_types=[
            pltpu.VMEM((EPW,), jnp.int32),
            pltpu.VMEM((CB, HS), jnp.float32),
            pltpu.VMEM((CB, HS), jnp.float32),
            pltpu.VMEM_SHARED((N, HS), jnp.float32),
            pltpu.SemaphoreType.DMA,
            pltpu.SemaphoreType.DMA,
        ],
    )
    def k(eh_h, dst_h, z_h, seh_o, idx_v, rows_a, rows_b, seh_sh,
          sem_a, sem_b):
        cid = lax.axis_index("c")
        sid = lax.axis_index("s")
        wid = cid * NS + sid
        _sliced_copy(z_h, seh_sh, sid)
        base_w = wid * EPW
        pltpu.sync_copy(dst_h.at[pl.ds(base_w, EPW)], idx_v)
        plsc.subcore_barrier()

        def load(j, buf, sem):
            return pltpu.async_copy(
                eh_h.at[pl.ds(base_w + j * CB, CB), pl.ds(0, HS)], buf, sem)

        def flush(j, buf):
            pltpu.sync_copy(buf, seh_sh.at[idx_v.at[pl.ds(j * CB, CB)]],
                            add=True)

        load(0, rows_a, sem_a).wait()

        def pair(i, carry):
            ja = 2 * i
            db = load(ja + 1, rows_b, sem_b)
            flush(ja, rows_a)
            da = load(ja + 2, rows_a, sem_a)
            db.wait()
            flush(ja + 1, rows_b)
            da.wait()
            return carry

        lax.fori_loop(0, NPAIR, pair, 0)
        flush(NCH - 1, rows_a)
        plsc.subcore_barrier()
        _sliced_copy(seh_sh, seh_o.at[cid], sid)

    return k(eh, dst3, zeros_hs)


def _sc_gather_segsum(h, src3, dst3, zeros_h):
    @functools.partial(
        pl.kernel,
        out_type=jax.ShapeDtypeStruct((NC, N, H), jnp.float32),
        mesh=_SC_MESH,
        compiler_params=_SC_PARAMS,
        scratch_types=[
            pltpu.VMEM((EPW,), jnp.int32),
            pltpu.VMEM((EPW,), jnp.int32),
            pltpu.VMEM((CB, H), jnp.float32),
            pltpu.VMEM((CB, H), jnp.float32),
            pltpu.VMEM_SHARED((N, H), jnp.float32),
            pltpu.SemaphoreType.DMA,
            pltpu.SemaphoreType.DMA,
        ],
    )
    def k(h_h, src_h, dst_h, z_h, s_o, idxs_v, idxd_v, rows_a, rows_b,
          acc_sh, sem_a, sem_b):
        cid = lax.axis_index("c")
        sid = lax.axis_index("s")
        wid = cid * NS + sid
        _sliced_copy(z_h, acc_sh, sid)
        base_w = wid * EPW
        pltpu.sync_copy(src_h.at[pl.ds(base_w, EPW)], idxs_v)
        pltpu.sync_copy(dst_h.at[pl.ds(base_w, EPW)], idxd_v)
        plsc.subcore_barrier()

        def gather(j, buf, sem):
            return pltpu.async_copy(h_h.at[idxs_v.at[pl.ds(j * CB, CB)]],
                                    buf, sem)

        def flush(j, buf):
            pltpu.sync_copy(buf, acc_sh.at[idxd_v.at[pl.ds(j * CB, CB)]],
                            add=True)

        gather(0, rows_a, sem_a).wait()

        def pair(i, carry):
            ja = 2 * i
            db = gather(ja + 1, rows_b, sem_b)
            flush(ja, rows_a)
            da = gather(ja + 2, rows_a, sem_a)
            db.wait()
            flush(ja + 1, rows_b)
            da.wait()
            return carry

        lax.fori_loop(0, NPAIR, pair, 0)
        flush(NCH - 1, rows_a)
        plsc.subcore_barrier()
        _sliced_copy(acc_sh, s_o.at[cid], sid)

    return k(h, src3, dst3, zeros_h)


def _sc_gather_z(z, src3, dst3):
    @functools.partial(
        pl.kernel,
        out_type=(
            jax.ShapeDtypeStruct((NW * NCH, CB, L), jnp.float32),
            jax.ShapeDtypeStruct((NW * NCH, CB, L), jnp.float32),
        ),
        mesh=_SC_MESH,
        compiler_params=_SC_PARAMS,
        scratch_types=[
            pltpu.VMEM((EPW,), jnp.int32),
            pltpu.VMEM((EPW,), jnp.int32),
            pltpu.VMEM((CB, L), jnp.float32),
            pltpu.VMEM((CB, L), jnp.float32),
            pltpu.VMEM((CB, L), jnp.float32),
            pltpu.VMEM((CB, L), jnp.float32),
            pltpu.SemaphoreType.DMA,
            pltpu.SemaphoreType.DMA,
        ],
    )
    def k(z_h, src_h, dst_h, gzs_o, gzd_o, idxs_v, idxd_v,
          rs_a, rd_a, rs_b, rd_b, sem_a, sem_b):
        cid = lax.axis_index("c")
        sid = lax.axis_index("s")
        wid = cid * NS + sid
        base_w = wid * EPW
        pltpu.sync_copy(src_h.at[pl.ds(base_w, EPW)], idxs_v)
        pltpu.sync_copy(dst_h.at[pl.ds(base_w, EPW)], idxd_v)

        def gathers(j, rs, rd, sem):
            sl = pl.ds(j * CB, CB)
            cs = pltpu.async_copy(z_h.at[idxs_v.at[sl]], rs, sem)
            cd = pltpu.async_copy(z_h.at[idxd_v.at[sl]], rd, sem)
            return cs, cd

        def flush(j, rs, rd):
            pltpu.sync_copy(rs, gzs_o.at[wid * NCH + j])
            pltpu.sync_copy(rd, gzd_o.at[wid * NCH + j])

        ca, cb2 = gathers(0, rs_a, rd_a, sem_a)
        ca.wait()
        cb2.wait()

        def pair(i, carry):
            ja = 2 * i
            b1, b2 = gathers(ja + 1, rs_b, rd_b, sem_b)
            flush(ja, rs_a, rd_a)
            a1, a2 = gathers(ja + 2, rs_a, rd_a, sem_a)
            b1.wait()
            b2.wait()
            flush(ja + 1, rs_b, rd_b)
            a1.wait()
            a2.wait()
            return carry

        lax.fori_loop(0, NPAIR, pair, 0)
        flush(NCH - 1, rs_a, rd_a)

    return k(z, src3, dst3)


def _tc_in_proj(x, w, b):
    BM = 2000

    def body(x_r, w_r, b_r, o_r):
        o_r[...] = jax.nn.relu(
            jnp.dot(x_r[...], w_r[...], preferred_element_type=jnp.float32)
            + b_r[...]
        )

    return pl.pallas_call(
        body,
        grid=(N // BM,),
        in_specs=[
            pl.BlockSpec((BM, DF), lambda i: (i, 0)),
            pl.BlockSpec((DF, H), lambda i: (0, 0)),
            pl.BlockSpec((1, H), lambda i: (0, 0)),
        ],
        out_specs=pl.BlockSpec((BM, H), lambda i: (i, 0)),
        out_shape=jax.ShapeDtypeStruct((N, H), jnp.float32),
    )(x, w, b)


def _tc_edge_proj(ea_t, w_pad, b_pad):
    BM = 6400

    def body(a_r, w_r, b_r, o_r):
        o_r[...] = jax.nn.relu(
            lax.dot_general(a_r[...], w_r[...], (((0,), (0,)), ((), ())),
                            preferred_element_type=jnp.float32)
            + b_r[...]
        )

    return pl.pallas_call(
        body,
        grid=(E // BM,),
        in_specs=[
            pl.BlockSpec((DE, BM), lambda i: (0, i)),
            pl.BlockSpec((DE, H2), lambda i: (0, 0)),
            pl.BlockSpec((1, H2), lambda i: (0, 0)),
        ],
        out_specs=pl.BlockSpec((BM, H2), lambda i: (i, 0)),
        out_shape=jax.ShapeDtypeStruct((E, H2), jnp.float32),
    )(ea_t, w_pad, b_pad)


def _tc_layer_combine(h, s, seh, ws, wn, b):
    def body(h_r, s_r, seh_r, ws_r, wn_r, b_r, o_r):
        deg = seh_r[0, :, H:H + 1] + seh_r[1, :, H:H + 1]
        deg = jnp.maximum(deg, 1.0)
        agg = (s_r[0] + s_r[1] + seh_r[0, :, :H] + seh_r[1, :, :H]) / deg
        o_r[...] = jax.nn.relu(
            jnp.dot(h_r[...], ws_r[...], preferred_element_type=jnp.float32)
            + jnp.dot(agg, wn_r[...], preferred_element_type=jnp.float32)
            + b_r[...]
        )

    return pl.pallas_call(
        body,
        out_shape=jax.ShapeDtypeStruct((N, H), jnp.float32),
    )(h, s, seh, ws, wn, b)


def _tc_latent_node(h, w_lat, b_lat, nd_w1, nd_b1, nd_w2, nd_b2):
    def body(h_r, wl_r, bl_r, w1_r, b1_r, w2_r, b2_r, z_o, g_o, rn_o):
        z = jnp.dot(h_r[...], wl_r[...], preferred_element_type=jnp.float32) + bl_r[...]
        z_o[...] = z
        g_o[...] = jnp.mean(z, axis=0, keepdims=True)
        hid = jax.nn.relu(
            jnp.dot(z, w1_r[...], preferred_element_type=jnp.float32) + b1_r[...]
        )
        rn_o[...] = (
            jnp.dot(hid, w2_r[...], preferred_element_type=jnp.float32) + b2_r[...]
        )

    return pl.pallas_call(
        body,
        out_shape=(
            jax.ShapeDtypeStruct((N, L), jnp.float32),
            jax.ShapeDtypeStruct((1, L), jnp.float32),
            jax.ShapeDtypeStruct((N, DF), jnp.float32),
        ),
    )(h, w_lat, b_lat, nd_w1, nd_b1, nd_w2, nd_b2)


def _tc_edge_decode(gzs4, gzd4, w1a4, w1b4, b1_4, w2_4, b2_4):
    BM4 = 4000

    def body(s_r, d_r, w1a_r, w1b_r, b1_r, w2_r, b2_r, o_r):
        hid = jax.nn.relu(
            jnp.dot(s_r[...], w1a_r[...], preferred_element_type=jnp.float32)
            + jnp.dot(d_r[...], w1b_r[...], preferred_element_type=jnp.float32)
            + b1_r[...]
        )
        o_r[...] = (
            jnp.dot(hid, w2_r[...], preferred_element_type=jnp.float32) + b2_r[...]
        )

    return pl.pallas_call(
        body,
        grid=(E // 4 // BM4,),
        in_specs=[
            pl.BlockSpec((BM4, 4 * L), lambda i: (i, 0)),
            pl.BlockSpec((BM4, 4 * L), lambda i: (i, 0)),
            pl.BlockSpec((4 * L, 4 * H), lambda i: (0, 0)),
            pl.BlockSpec((4 * L, 4 * H), lambda i: (0, 0)),
            pl.BlockSpec((1, 4 * H), lambda i: (0, 0)),
            pl.BlockSpec((4 * H, 4 * DE), lambda i: (0, 0)),
            pl.BlockSpec((1, 4 * DE), lambda i: (0, 0)),
        ],
        out_specs=pl.BlockSpec((BM4, 4 * DE), lambda i: (i, 0)),
        out_shape=jax.ShapeDtypeStruct((E // 4, 4 * DE), jnp.float32),
    )(gzs4, gzd4, w1a4, w1b4, b1_4, w2_4, b2_4)


def kernel(x, edge_index, edge_attr, W_in, b_in, W_edge, b_edge, W_self0,
           W_nbr0, b0, W_self1, W_nbr1, b1, W_lat, b_lat, nd_W1, nd_b1,
           nd_W2, nd_b2, ed_W1, ed_b1, ed_W2, ed_b2):
    src = edge_index[0].astype(jnp.int32)
    dst = edge_index[1].astype(jnp.int32)

    zeros_h = jnp.zeros((N, H), jnp.float32)
    zeros_hs = jnp.zeros((N, HS), jnp.float32)

    w_edge_pad = jnp.zeros((DE, H2), jnp.float32).at[:, :H].set(W_edge)
    b_edge_pad = jnp.zeros((1, H2), jnp.float32).at[:, :H].set(b_edge)
    b_edge_pad = b_edge_pad.at[:, H].set(1.0)
    bd = jax.scipy.linalg.block_diag
    w1a4 = bd(*([ed_W1[:L]] * 4))
    w1b4 = bd(*([ed_W1[L:]] * 4))
    w2_4 = bd(*([ed_W2] * 4))
    b1_4 = jnp.tile(ed_b1, 4).reshape(1, 4 * H)
    b2_4 = jnp.tile(ed_b2, 4).reshape(1, 4 * DE)

    h = _tc_in_proj(x, W_in, b_in.reshape(1, H))
    eh = _tc_edge_proj(edge_attr.T, w_edge_pad, b_edge_pad)

    seh = _sc_edge_static(eh, dst, zeros_hs)

    for ws, wn, b in ((W_self0, W_nbr0, b0), (W_self1, W_nbr1, b1)):
        s = _sc_gather_segsum(h, src, dst, zeros_h)
        h = _tc_layer_combine(h, s, seh, ws, wn, b.reshape(1, H))

    z, g, recon_node = _tc_latent_node(
        h, W_lat, b_lat.reshape(1, L), nd_W1, nd_b1.reshape(1, H),
        nd_W2, nd_b2.reshape(1, DF))

    gzs, gzd = _sc_gather_z(z, src, dst)
    recon_edge4 = _tc_edge_decode(
        gzs.reshape(E // 4, 4 * L), gzd.reshape(E // 4, 4 * L),
        w1a4, w1b4, b1_4, w2_4, b2_4)

    recon_edge = recon_edge4.reshape(E // 4, 4, DE).transpose(2, 0, 1).reshape(DE, E).T

    return (z, g, recon_node, recon_edge)
